# baseline (speedup 1.0000x reference)
import functools

import jax
import jax.numpy as jnp
from jax import lax
from jax.experimental import pallas as pl
from jax.experimental.pallas import tpu as pltpu

B, N, D = 1024, 77, 768


BB = 256


def _select_body(pt_ref, tok_ref, emb_ref, ph_ref, out_ref):
    j = pl.program_id(0)
    tok = tok_ref[...]
    lane = lax.broadcasted_iota(jnp.int32, (BB, N), 1)
    hit = jnp.where((tok == pt_ref[0]) & (lane == j), 1, 0)
    col = jnp.max(hit, axis=1, keepdims=True)
    out_ref[0] = jnp.where(col == 1, ph_ref[0], emb_ref[0])


def kernel(tokenized_text, embedded_text, placeholder_embedding, placeholder_token):
    pt = placeholder_token.reshape((1,)).astype(tokenized_text.dtype)
    emb_t = embedded_text.transpose(1, 0, 2)
    ph3 = placeholder_embedding[None]
    out_t = pl.pallas_call(
        _select_body,
        grid_spec=pltpu.PrefetchScalarGridSpec(
            num_scalar_prefetch=1,
            grid=(N, B // BB),
            in_specs=[
                pl.BlockSpec((BB, N), lambda j, i, pt: (i, 0)),
                pl.BlockSpec((1, BB, D), lambda j, i, pt: (j, i, 0)),
                pl.BlockSpec((1, 1, D), lambda j, i, pt: (0, 0, 0)),
            ],
            out_specs=pl.BlockSpec((1, BB, D), lambda j, i, pt: (j, i, 0)),
        ),
        out_shape=jax.ShapeDtypeStruct((N, B, D), jnp.float32),
        compiler_params=pltpu.CompilerParams(
            dimension_semantics=("arbitrary", "arbitrary"),
        ),
    )(pt, tokenized_text, emb_t, ph3)
    return out_t.transpose(1, 0, 2)


# speedup vs baseline: 1.4896x; 1.4896x over previous
import functools

import jax
import jax.numpy as jnp
from jax import lax
from jax.experimental import pallas as pl
from jax.experimental.pallas import tpu as pltpu
from jax.experimental.pallas import tpu_sc as plsc

B, N, D = 1024, 77, 768
R = B * N
NC, NS = 2, 16
NW = NC * NS
CPW = R // NW
NV = CPW // 16
CH = 56
NCH = CPW // CH


def _sc_body(tok_hbm, pt_hbm, ph_hbm, emb_hbm, out_hbm,
             tok_v, pt_v, pay, buf0, buf1,
             sin0, sin1, sout0, sout1, smisc):
    wid = lax.axis_index("s") * NC + lax.axis_index("c")
    base = wid * CPW

    pltpu.sync_copy(tok_hbm.at[pl.ds(base, CPW)], tok_v)
    pltpu.sync_copy(pt_hbm, pt_v)
    pltpu.sync_copy(ph_hbm, pay.at[0])
    ptv = pt_v[...]

    bufs = (buf0, buf1)
    sins = (sin0, sin1)
    souts = (sout0, sout1)
    for b in range(2):
        pltpu.async_copy(emb_hbm.at[pl.ds(base + b * CH, CH)], bufs[b], sins[b])

    def chunk_body(g2, carry):
        for b in range(2):
            g = g2 * 2 + b
            row0 = base + g * CH
            pltpu.make_async_copy(emb_hbm.at[pl.ds(row0, CH)], bufs[b], sins[b]).wait()
            pltpu.async_copy(bufs[b], out_hbm.at[pl.ds(row0, CH)], souts[b])

            @pl.when(g + 2 < NCH)
            def _():
                pltpu.make_async_copy(bufs[b], out_hbm.at[pl.ds(row0, CH)], souts[b]).wait()
                pltpu.async_copy(emb_hbm.at[pl.ds(base + (g + 2) * CH, CH)], bufs[b], sins[b])
        return carry

    lax.fori_loop(0, NCH // 2, chunk_body, 0)
    for b in range(2):
        g_last = NCH - 2 + b
        pltpu.make_async_copy(
            bufs[b], out_hbm.at[pl.ds(base + g_last * CH, CH)], souts[b]).wait()

    def scan_body(i, carry):
        tv = tok_v[pl.ds(i * 16, 16)]
        mi = jnp.where(tv == ptv, 1, 0)
        for l in range(16):
            @pl.when(mi[l] == 1)
            def _():
                pltpu.async_copy(pay.at[0], out_hbm.at[base + i * 16 + l], smisc).wait()
        return carry

    lax.fori_loop(0, NV, scan_body, 0)


def kernel(tokenized_text, embedded_text, placeholder_embedding, placeholder_token):
    tok_flat = tokenized_text.T.reshape(R)
    pt16 = jnp.full((16,), placeholder_token, tokenized_text.dtype)
    ph_row = placeholder_embedding.reshape(D)
    emb_flat = embedded_text.transpose(1, 0, 2).reshape(R, D)

    run = functools.partial(
        pl.kernel,
        out_type=jax.ShapeDtypeStruct((R, D), jnp.float32),
        mesh=plsc.VectorSubcoreMesh(core_axis_name="c", subcore_axis_name="s"),
        scratch_types=[
            pltpu.VMEM((CPW,), jnp.int32),
            pltpu.VMEM((16,), jnp.int32),
            pltpu.VMEM((1, D), jnp.float32),
            pltpu.VMEM((CH, D), jnp.float32),
            pltpu.VMEM((CH, D), jnp.float32),
            pltpu.SemaphoreType.DMA,
            pltpu.SemaphoreType.DMA,
            pltpu.SemaphoreType.DMA,
            pltpu.SemaphoreType.DMA,
            pltpu.SemaphoreType.DMA,
        ],
    )(_sc_body)
    out_flat = run(tok_flat, pt16, ph_row, emb_flat)
    return out_flat.reshape(N, B, D).transpose(1, 0, 2)


# speedup vs baseline: 1.4967x; 1.0048x over previous
import functools

import jax
import jax.numpy as jnp
from jax import lax
from jax.experimental import pallas as pl
from jax.experimental.pallas import tpu as pltpu
from jax.experimental.pallas import tpu_sc as plsc

B, N, D = 1024, 77, 768
R = B * N
NC, NS = 2, 16
NW = NC * NS
CPW = R // NW
NV = CPW // 16
CH = 16
NCH = CPW // CH
NBUF = 7


def _sc_body(tok_hbm, pt_hbm, ph_hbm, emb_hbm, out_hbm,
             tok_v, pt_v, pay, bufs, sins, souts, smisc):
    wid = lax.axis_index("s") * NC + lax.axis_index("c")
    base = wid * CPW

    pltpu.sync_copy(tok_hbm.at[pl.ds(base, CPW)], tok_v)
    pltpu.sync_copy(pt_hbm, pt_v)
    pltpu.sync_copy(ph_hbm, pay.at[0])
    ptv = pt_v[...]

    for b in range(NBUF):
        pltpu.async_copy(emb_hbm.at[pl.ds(base + b * CH, CH)], bufs[b], sins[b])

    def chunk_body(g2, carry):
        for b in range(NBUF):
            g = g2 * NBUF + b
            row0 = base + g * CH
            pltpu.make_async_copy(emb_hbm.at[pl.ds(row0, CH)], bufs[b], sins[b]).wait()
            pltpu.async_copy(bufs[b], out_hbm.at[pl.ds(row0, CH)], souts[b])

            @pl.when(g + NBUF < NCH)
            def _():
                pltpu.make_async_copy(bufs[b], out_hbm.at[pl.ds(row0, CH)], souts[b]).wait()
                pltpu.async_copy(emb_hbm.at[pl.ds(base + (g + NBUF) * CH, CH)], bufs[b], sins[b])
        return carry

    lax.fori_loop(0, NCH // NBUF, chunk_body, 0)
    for b in range(NBUF):
        g_last = NCH - NBUF + b
        pltpu.make_async_copy(
            bufs[b], out_hbm.at[pl.ds(base + g_last * CH, CH)], souts[b]).wait()

    def scan_body(i, carry):
        tv = tok_v[pl.ds(i * 16, 16)]
        mi = jnp.where(tv == ptv, 1, 0)
        for l in range(16):
            @pl.when(mi[l] == 1)
            def _():
                pltpu.async_copy(pay.at[0], out_hbm.at[base + i * 16 + l], smisc).wait()
        return carry

    lax.fori_loop(0, NV, scan_body, 0)


def kernel(tokenized_text, embedded_text, placeholder_embedding, placeholder_token):
    tok_flat = tokenized_text.T.reshape(R)
    pt16 = jnp.full((16,), placeholder_token, tokenized_text.dtype)
    ph_row = placeholder_embedding.reshape(D)
    emb_flat = embedded_text.transpose(1, 0, 2).reshape(R, D)

    run = functools.partial(
        pl.kernel,
        out_type=jax.ShapeDtypeStruct((R, D), jnp.float32),
        mesh=plsc.VectorSubcoreMesh(core_axis_name="c", subcore_axis_name="s"),
        scratch_types=[
            pltpu.VMEM((CPW,), jnp.int32),
            pltpu.VMEM((16,), jnp.int32),
            pltpu.VMEM((1, D), jnp.float32),
            [pltpu.VMEM((CH, D), jnp.float32) for _ in range(NBUF)],
            [pltpu.SemaphoreType.DMA for _ in range(NBUF)],
            [pltpu.SemaphoreType.DMA for _ in range(NBUF)],
            pltpu.SemaphoreType.DMA,
        ],
    )(_sc_body)
    out_flat = run(tok_flat, pt16, ph_row, emb_flat)
    return out_flat.reshape(N, B, D).transpose(1, 0, 2)
